# baseline (speedup 1.0000x reference)
import jax
import jax.numpy as jnp
from jax.experimental import pallas as pl
from jax.experimental.pallas import tpu as pltpu


def _clam_block(h_ref, W1_ref, b1_ref, Wab_ref, bab_ref, wc_ref, bc_ref,
                Wcls_ref, bcls_ref,
                A_ref, logits_ref, yprob_ref, yhat_ref,
                m_ref, s_ref, acc_ref):
    j = pl.program_id(0)
    nblk = pl.num_programs(0)
    d_att = wc_ref.shape[1]
    d_in = h_ref.shape[1]
    khalf = d_in // 2

    @pl.when(j == 0)
    def _init():
        m_ref[...] = (jnp.sum(jnp.abs(wc_ref[...]), axis=1, keepdims=True)
                      + jnp.abs(bc_ref[...]))
        s_ref[...] = jnp.zeros_like(s_ref)
        acc_ref[...] = jnp.zeros_like(acc_ref)

    h_blk = h_ref[...]
    h1f = jnp.dot(h_blk, W1_ref[...], preferred_element_type=jnp.float32)
    h1 = jnp.maximum(h1f + b1_ref[...], 0.0).astype(jnp.bfloat16)
    ab = jnp.dot(h1, Wab_ref[...], preferred_element_type=jnp.float32)
    ab = ab + bab_ref[...]
    a = jnp.tanh(ab[:, :d_att])
    b = jax.nn.sigmoid(ab[:, d_att:])
    g = a * b
    att = jnp.sum(g * wc_ref[...], axis=1, keepdims=True) + bc_ref[...]
    A_ref[...] = att

    p = jnp.exp(att - m_ref[...])
    s_ref[...] = s_ref[...] + jnp.sum(p, axis=(0, 1), keepdims=True)
    pw = jax.lax.dot_general(p.astype(jnp.bfloat16), h1,
                             (((0,), (0,)), ((), ())),
                             preferred_element_type=jnp.float32)
    acc_ref[...] = acc_ref[...] + pw

    @pl.when(j == nblk - 1)
    def _head():
        M = acc_ref[...] / s_ref[...]
        logits = jnp.dot(M, Wcls_ref[...], preferred_element_type=jnp.float32)
        logits = logits + bcls_ref[...]
        logits_ref[...] = logits
        mx = jnp.max(logits, axis=1, keepdims=True)
        e = jnp.exp(logits - mx)
        yprob_ref[...] = e / jnp.sum(e, axis=1, keepdims=True)
        c = logits.shape[1]
        idx = jax.lax.broadcasted_iota(jnp.int32, logits.shape, 1)
        yhat_ref[...] = jnp.min(jnp.where(logits == mx, idx, c), axis=1,
                                keepdims=True)


def kernel(h, W1, b1, Wa, ba, Wb, bb, Wc, bc, Wcls, bcls):
    n, d_in = h.shape
    d_hid = W1.shape[1]
    d_att = Wa.shape[1]
    n_classes = Wcls.shape[1]

    blk = 2000
    if n % blk != 0:
        blk = next(b for b in (500, 250, 200, 100, 50, 25, 10, 8, 5, 4, 2, 1)
                   if n % b == 0)
    nblk = n // blk

    W1_b = W1
    Wab_b = jnp.concatenate([Wa, Wb], axis=1).astype(jnp.bfloat16)
    b1_r = b1.reshape(1, d_hid)
    bab_r = jnp.concatenate([ba, bb]).reshape(1, 2 * d_att)
    wc_r = Wc.reshape(1, d_att)
    bc_r = bc.reshape(1, 1)
    bcls_r = bcls.reshape(1, n_classes)

    const = lambda j: (0, 0)
    A_col, logits, y_prob, y_hat = pl.pallas_call(
        _clam_block,
        grid=(nblk,),
        in_specs=[
            pl.BlockSpec((blk, d_in), lambda j: (j, 0)),
            pl.BlockSpec((d_in, d_hid), const),
            pl.BlockSpec((1, d_hid), const),
            pl.BlockSpec((d_hid, 2 * d_att), const),
            pl.BlockSpec((1, 2 * d_att), const),
            pl.BlockSpec((1, d_att), const),
            pl.BlockSpec((1, 1), const),
            pl.BlockSpec((d_hid, n_classes), const),
            pl.BlockSpec((1, n_classes), const),
        ],
        out_specs=[
            pl.BlockSpec((blk, 1), lambda j: (j, 0)),
            pl.BlockSpec((1, n_classes), const),
            pl.BlockSpec((1, n_classes), const),
            pl.BlockSpec((1, 1), const),
        ],
        out_shape=[
            jax.ShapeDtypeStruct((n, 1), jnp.float32),
            jax.ShapeDtypeStruct((1, n_classes), jnp.float32),
            jax.ShapeDtypeStruct((1, n_classes), jnp.float32),
            jax.ShapeDtypeStruct((1, 1), jnp.int32),
        ],
        scratch_shapes=[
            pltpu.VMEM((1, 1), jnp.float32),
            pltpu.VMEM((1, 1), jnp.float32),
            pltpu.VMEM((1, d_hid), jnp.float32),
        ],
        compiler_params=pltpu.CompilerParams(
            dimension_semantics=("arbitrary",),
        ),
    )(h, W1_b, b1_r, Wab_b, bab_r, wc_r, bc_r, Wcls, bcls_r)

    return (logits, y_prob, y_hat, A_col.reshape(1, n))


# speedup vs baseline: 1.0017x; 1.0017x over previous
import jax
import jax.numpy as jnp
from jax.experimental import pallas as pl
from jax.experimental.pallas import tpu as pltpu


def _clam_block(h_ref, W1_ref, b1_ref, Wab_ref, bab_ref, wc_ref, bc_ref,
                Wcls_ref, bcls_ref,
                A_ref, logits_ref, yprob_ref, yhat_ref,
                m_ref, s_ref, acc_ref):
    j = pl.program_id(0)
    nblk = pl.num_programs(0)
    d_att = wc_ref.shape[1]

    @pl.when(j == 0)
    def _init():
        m_ref[...] = (jnp.sum(jnp.abs(wc_ref[...]), axis=1, keepdims=True)
                      + jnp.abs(bc_ref[...]))
        s_ref[...] = jnp.zeros_like(s_ref)
        acc_ref[...] = jnp.zeros_like(acc_ref)

    h_blk = h_ref[...]
    h1f = jnp.dot(h_blk, W1_ref[...], preferred_element_type=jnp.float32)
    h1 = jnp.maximum(h1f + b1_ref[...], 0.0).astype(jnp.bfloat16)
    ab = jnp.dot(h1, Wab_ref[...], preferred_element_type=jnp.float32)
    ab = ab + bab_ref[...]
    a = jnp.tanh(ab[:, :d_att])
    b = jax.nn.sigmoid(ab[:, d_att:])
    g = a * b
    att = jnp.sum(g * wc_ref[...], axis=1, keepdims=True) + bc_ref[...]
    A_ref[...] = att

    p = jnp.exp(att - m_ref[...])
    s_ref[...] = s_ref[...] + jnp.sum(p, axis=(0, 1), keepdims=True)
    pw = jax.lax.dot_general(p.astype(jnp.bfloat16), h1,
                             (((0,), (0,)), ((), ())),
                             preferred_element_type=jnp.float32)
    acc_ref[...] = acc_ref[...] + pw

    @pl.when(j == nblk - 1)
    def _head():
        M = acc_ref[...] / s_ref[...]
        logits = jnp.dot(M, Wcls_ref[...], preferred_element_type=jnp.float32)
        logits = logits + bcls_ref[...]
        logits_ref[...] = logits
        mx = jnp.max(logits, axis=1, keepdims=True)
        e = jnp.exp(logits - mx)
        yprob_ref[...] = e / jnp.sum(e, axis=1, keepdims=True)
        c = logits.shape[1]
        idx = jax.lax.broadcasted_iota(jnp.int32, logits.shape, 1)
        yhat_ref[...] = jnp.min(jnp.where(logits == mx, idx, c), axis=1,
                                keepdims=True)


def kernel(h, W1, b1, Wa, ba, Wb, bb, Wc, bc, Wcls, bcls):
    n, d_in = h.shape
    d_hid = W1.shape[1]
    d_att = Wa.shape[1]
    n_classes = Wcls.shape[1]

    blk = next((b for b in (2000, 1000, 400, 200, 104, 56, 8) if n % b == 0),
               n)
    nblk = n // blk

    Wab_b = jnp.concatenate([Wa, Wb], axis=1).astype(jnp.bfloat16)
    b1_r = b1.reshape(1, d_hid)
    bab_r = jnp.concatenate([ba, bb]).reshape(1, 2 * d_att)
    wc_r = Wc.reshape(1, d_att)
    bc_r = bc.reshape(1, 1)
    bcls_r = bcls.reshape(1, n_classes)

    const = lambda j: (0, 0)
    A_col, logits, y_prob, y_hat = pl.pallas_call(
        _clam_block,
        grid=(nblk,),
        in_specs=[
            pl.BlockSpec((blk, d_in), lambda j: (j, 0)),
            pl.BlockSpec((d_in, d_hid), const),
            pl.BlockSpec((1, d_hid), const),
            pl.BlockSpec((d_hid, 2 * d_att), const),
            pl.BlockSpec((1, 2 * d_att), const),
            pl.BlockSpec((1, d_att), const),
            pl.BlockSpec((1, 1), const),
            pl.BlockSpec((d_hid, n_classes), const),
            pl.BlockSpec((1, n_classes), const),
        ],
        out_specs=[
            pl.BlockSpec((blk, 1), lambda j: (j, 0)),
            pl.BlockSpec((1, n_classes), const),
            pl.BlockSpec((1, n_classes), const),
            pl.BlockSpec((1, 1), const),
        ],
        out_shape=[
            jax.ShapeDtypeStruct((n, 1), jnp.float32),
            jax.ShapeDtypeStruct((1, n_classes), jnp.float32),
            jax.ShapeDtypeStruct((1, n_classes), jnp.float32),
            jax.ShapeDtypeStruct((1, 1), jnp.int32),
        ],
        scratch_shapes=[
            pltpu.VMEM((1, 1), jnp.float32),
            pltpu.VMEM((1, 1), jnp.float32),
            pltpu.VMEM((1, d_hid), jnp.float32),
        ],
        compiler_params=pltpu.CompilerParams(
            dimension_semantics=("arbitrary",),
        ),
    )(h, W1, b1_r, Wab_b, bab_r, wc_r, bc_r, Wcls, bcls_r)

    return (logits, y_prob, y_hat, A_col.reshape(1, n))
